# Initial kernel scaffold; baseline (speedup 1.0000x reference)
import jax
import jax.numpy as jnp
from jax.experimental import pallas as pl


def kernel(x, edge_attr, edge_index, W_atom, b_atom, W_bond0, b_bond0, g0, be0, W_bondA, b_bondA, W1A, b1A, W2A, b2A, g1A, be1A, g2A, be2A, W_bondB, b_bondB, W1B, b1B, W2B, b2B, g1B, be1B, g2B, be2B, W_last, b_last, gl, bel):
    raise NotImplementedError("write your pallas kernel here")


# speedup vs baseline: 3.4614x; 3.4614x over previous
import functools

import jax
import jax.numpy as jnp
from jax import lax
from jax.experimental import pallas as pl
from jax.experimental.pallas import tpu as pltpu
from jax.experimental.pallas import tpu_sc as plsc

N = 10000
E = 320000
D = 128
DE = 16

NC = 2
NS = 16
NW = NC * NS
K = 128
EPT = 10240
NCH = EPT // K
EP = NW * EPT
NACC = 10240
ZPT = NACC // NS

_mesh = plsc.VectorSubcoreMesh(
    core_axis_name="c", subcore_axis_name="s", num_cores=NC, num_subcores=NS)


_EA_OUT = jax.ShapeDtypeStruct((NC, NACC, D), jnp.float32)
_EA_SCRATCH = [
    pltpu.VMEM((K, D), jnp.float32),
    pltpu.VMEM_SHARED((NACC, D), jnp.float32),
    pltpu.VMEM((K,), jnp.int32),
]


def _ea_body(ea_hbm, dst_hbm, out_hbm, rows_v, acc_s, didx_v):
    c = lax.axis_index("c")
    s = lax.axis_index("s")
    wid = c * NS + s

    def zrow(i, _):
        def zcol(j, _):
            rows_v[i, pl.ds(j * 16, 16)] = jnp.zeros((16,), jnp.float32)
            return 0
        return lax.fori_loop(0, D // 16, zcol, 0)
    lax.fori_loop(0, K, zrow, 0)

    def zero_chunk(z, _):
        pltpu.sync_copy(rows_v, acc_s.at[pl.ds(s * ZPT + z * K, K)])
        return 0
    lax.fori_loop(0, ZPT // K, zero_chunk, 0)
    plsc.subcore_barrier()

    def body(ch, _):
        pltpu.sync_copy(dst_hbm.at[wid, ch], didx_v)
        pltpu.sync_copy(ea_hbm.at[pl.ds(wid * EPT + ch * K, K)], rows_v)
        pltpu.sync_copy(rows_v, acc_s.at[didx_v], add=True)
        return 0
    lax.fori_loop(0, NCH, body, 0)
    plsc.subcore_barrier()

    pltpu.sync_copy(acc_s.at[pl.ds(s * ZPT, ZPT)],
                    out_hbm.at[c, pl.ds(s * ZPT, ZPT)])


_ea_kernel = pl.kernel(
    _ea_body, out_type=_EA_OUT, mesh=_mesh, scratch_types=_EA_SCRATCH)


_MSG_OUT = jax.ShapeDtypeStruct((NC, NACC, D), jnp.float32)
_MSG_SCRATCH = [
    pltpu.VMEM((K, D), jnp.float32),
    pltpu.VMEM_SHARED((NACC, D), jnp.float32),
    pltpu.VMEM((K,), jnp.int32),
    pltpu.VMEM((K,), jnp.int32),
    pltpu.SemaphoreType.DMA,
]


def _msg_body(h_hbm, src_hbm, dst_hbm, out_hbm,
              rows_v, acc_s, sidx_v, didx_v, sem):
    c = lax.axis_index("c")
    s = lax.axis_index("s")
    wid = c * NS + s

    def zrow(i, _):
        def zcol(j, _):
            rows_v[i, pl.ds(j * 16, 16)] = jnp.zeros((16,), jnp.float32)
            return 0
        return lax.fori_loop(0, D // 16, zcol, 0)
    lax.fori_loop(0, K, zrow, 0)

    def zero_chunk(z, _):
        pltpu.sync_copy(rows_v, acc_s.at[pl.ds(s * ZPT + z * K, K)])
        return 0
    lax.fori_loop(0, ZPT // K, zero_chunk, 0)
    plsc.subcore_barrier()

    def body(ch, _):
        pltpu.sync_copy(src_hbm.at[wid, ch], sidx_v)
        pltpu.sync_copy(dst_hbm.at[wid, ch], didx_v)
        pltpu.async_copy(h_hbm.at[sidx_v], rows_v, sem).wait()
        pltpu.sync_copy(rows_v, acc_s.at[didx_v], add=True)
        return 0
    lax.fori_loop(0, NCH, body, 0)
    plsc.subcore_barrier()

    pltpu.sync_copy(acc_s.at[pl.ds(s * ZPT, ZPT)],
                    out_hbm.at[c, pl.ds(s * ZPT, ZPT)])


_msg_kernel = pl.kernel(
    _msg_body, out_type=_MSG_OUT, mesh=_mesh, scratch_types=_MSG_SCRATCH)


def _bn(y, g, b):
    m = jnp.mean(y, axis=0, keepdims=True)
    v = jnp.mean((y - m) ** 2, axis=0, keepdims=True)
    return (y - m) / jnp.sqrt(v + 1e-5) * g + b


def _dot(a, b):
    return jnp.dot(a, b, preferred_element_type=jnp.float32)


def _dense0_body(x_ref, eap_ref,
                 Wa, ba, Wb0, bb0, g0, be0, WbA, bbA, WbB, bbB,
                 h_ref, h2a_ref, h2b_ref):
    eadeg = eap_ref[0, :N] + eap_ref[1, :N]
    ea = eadeg[:, 0:DE]
    deg = eadeg[:, DE:DE + 1]
    y = _dot(x_ref[...], Wa[...]) + ba[...]
    y = y + _dot(ea, Wb0[...]) + deg * bb0[...]
    h_ref[...] = jnp.maximum(_bn(y, g0[...], be0[...]), 0.0)
    h2a_ref[...] = _dot(ea, WbA[...]) + deg * bbA[...]
    h2b_ref[...] = _dot(ea, WbB[...]) + deg * bbB[...]


def _dense_mid_body(h1p_ref, h2_ref, h_ref,
                    W1, b1, g1, be1, W2, b2, g2, be2, o_ref):
    h1 = h1p_ref[0, :N] + h1p_ref[1, :N]
    t = jnp.maximum(
        _bn(_dot(h1, W1[...]) + b1[...] + h2_ref[...], g1[...], be1[...]), 0.0)
    o_ref[...] = jnp.maximum(
        _bn(_dot(t, W2[...]) + b2[...] + h_ref[...], g2[...], be2[...]), 0.0)


def _dense_last_body(h1p_ref, h2_ref, h_ref,
                     W1, b1, g1, be1, W2, b2, g2, be2,
                     Wl, bl, gl, bel, o_ref):
    h1 = h1p_ref[0, :N] + h1p_ref[1, :N]
    t = jnp.maximum(
        _bn(_dot(h1, W1[...]) + b1[...] + h2_ref[...], g1[...], be1[...]), 0.0)
    hb = jnp.maximum(
        _bn(_dot(t, W2[...]) + b2[...] + h_ref[...], g2[...], be2[...]), 0.0)
    o_ref[...] = _bn(_dot(hb, Wl[...]) + bl[...], gl[...], bel[...])


_f32 = jnp.float32
_dense0 = pl.pallas_call(
    _dense0_body,
    out_shape=(jax.ShapeDtypeStruct((N, D), _f32),
               jax.ShapeDtypeStruct((N, D), _f32),
               jax.ShapeDtypeStruct((N, D), _f32)))
_dense_mid = pl.pallas_call(
    _dense_mid_body, out_shape=jax.ShapeDtypeStruct((N, D), _f32))
_dense_last = pl.pallas_call(
    _dense_last_body, out_shape=jax.ShapeDtypeStruct((N, D), _f32))


def kernel(x, edge_attr, edge_index,
           W_atom, b_atom, W_bond0, b_bond0, g0, be0,
           W_bondA, b_bondA, W1A, b1A, W2A, b2A, g1A, be1A, g2A, be2A,
           W_bondB, b_bondB, W1B, b1B, W2B, b2B, g1B, be1B, g2B, be2B,
           W_last, b_last, gl, bel):
    pad = EP - E
    src = jnp.concatenate(
        [edge_index[0], jnp.zeros((pad,), jnp.int32)]).reshape(NW, NCH, K)
    dst = jnp.concatenate(
        [edge_index[1], jnp.full((pad,), N, jnp.int32)]).reshape(NW, NCH, K)
    ea128 = jnp.concatenate(
        [edge_attr, jnp.ones((E, 1), jnp.float32),
         jnp.zeros((E, D - DE - 1), jnp.float32)], axis=1)
    ea128 = jnp.concatenate(
        [ea128, jnp.zeros((pad, D), jnp.float32)], axis=0)

    ea_parts = _ea_kernel(ea128, dst)

    r = lambda v: v.reshape(1, D)
    h, h2a, h2b = _dense0(
        x, ea_parts,
        W_atom, r(b_atom), W_bond0, r(b_bond0), r(g0), r(be0),
        W_bondA, r(b_bondA), W_bondB, r(b_bondB))

    h1p = _msg_kernel(h, src, dst)
    h = _dense_mid(h1p, h2a, h,
                   W1A, r(b1A), r(g1A), r(be1A), W2A, r(b2A), r(g2A), r(be2A))

    h1p = _msg_kernel(h, src, dst)
    out = _dense_last(h1p, h2b, h,
                      W1B, r(b1B), r(g1B), r(be1B), W2B, r(b2B), r(g2B), r(be2B),
                      W_last, r(b_last), r(gl), r(bel))
    return out


# speedup vs baseline: 4.3327x; 1.2517x over previous
import functools

import jax
import jax.numpy as jnp
from jax import lax
from jax.experimental import pallas as pl
from jax.experimental.pallas import tpu as pltpu
from jax.experimental.pallas import tpu_sc as plsc

N = 10000
E = 320000
D = 128
DE = 16

NC = 2
NS = 16
NW = NC * NS
K = 128
EPT = 10240
NCH = EPT // K
EP = NW * EPT
NACC = 10240
ZPT = NACC // NS

_mesh = plsc.VectorSubcoreMesh(
    core_axis_name="c", subcore_axis_name="s", num_cores=NC, num_subcores=NS)


_EA_OUT = jax.ShapeDtypeStruct((NC, NACC, D), jnp.float32)
_EA_SCRATCH = [
    pltpu.VMEM((K, D), jnp.float32),
    pltpu.VMEM((K, D), jnp.float32),
    pltpu.VMEM_SHARED((NACC, D), jnp.float32),
    pltpu.VMEM((K,), jnp.int32),
    pltpu.SemaphoreType.DMA,
    pltpu.SemaphoreType.DMA,
]


def _ea_body(ea_hbm, dst_hbm, out_hbm, rows0, rows1, acc_s, didx_v,
             sem0, sem1):
    c = lax.axis_index("c")
    s = lax.axis_index("s")
    wid = c * NS + s

    def zrow(i, _):
        def zcol(j, _):
            rows0[i, pl.ds(j * 16, 16)] = jnp.zeros((16,), jnp.float32)
            return 0
        return lax.fori_loop(0, D // 16, zcol, 0)
    lax.fori_loop(0, K, zrow, 0)

    def zero_chunk(z, _):
        pltpu.sync_copy(rows0, acc_s.at[pl.ds(s * ZPT + z * K, K)])
        return 0
    lax.fori_loop(0, ZPT // K, zero_chunk, 0)
    plsc.subcore_barrier()

    base = wid * EPT
    pltpu.async_copy(ea_hbm.at[pl.ds(base, K)], rows0, sem0)

    def pair(i, _):
        ch2 = i * 2
        pltpu.async_copy(ea_hbm.at[pl.ds(base + (ch2 + 1) * K, K)],
                         rows1, sem1)
        pltpu.make_async_copy(ea_hbm.at[pl.ds(0, K)], rows0, sem0).wait()
        pltpu.sync_copy(dst_hbm.at[wid, ch2], didx_v)
        pltpu.sync_copy(rows0, acc_s.at[didx_v], add=True)

        @pl.when(ch2 + 2 < NCH)
        def _():
            pltpu.async_copy(ea_hbm.at[pl.ds(base + (ch2 + 2) * K, K)],
                             rows0, sem0)
        pltpu.make_async_copy(ea_hbm.at[pl.ds(0, K)], rows1, sem1).wait()
        pltpu.sync_copy(dst_hbm.at[wid, ch2 + 1], didx_v)
        pltpu.sync_copy(rows1, acc_s.at[didx_v], add=True)
        return 0
    lax.fori_loop(0, NCH // 2, pair, 0)
    plsc.subcore_barrier()

    pltpu.sync_copy(acc_s.at[pl.ds(s * ZPT, ZPT)],
                    out_hbm.at[c, pl.ds(s * ZPT, ZPT)])


_ea_kernel = pl.kernel(
    _ea_body, out_type=_EA_OUT, mesh=_mesh, scratch_types=_EA_SCRATCH)


_MSG_OUT = jax.ShapeDtypeStruct((NC, NACC, D), jnp.float32)
_MSG_SCRATCH = [
    pltpu.VMEM((K, D), jnp.float32),
    pltpu.VMEM((K, D), jnp.float32),
    pltpu.VMEM_SHARED((NACC, D), jnp.float32),
    pltpu.VMEM((K,), jnp.int32),
    pltpu.VMEM((K,), jnp.int32),
    pltpu.VMEM((K,), jnp.int32),
    pltpu.SemaphoreType.DMA,
    pltpu.SemaphoreType.DMA,
]


def _msg_body(h_hbm, src_hbm, dst_hbm, out_hbm,
              rows0, rows1, acc_s, sidx0, sidx1, didx_v, sem0, sem1):
    c = lax.axis_index("c")
    s = lax.axis_index("s")
    wid = c * NS + s

    def zrow(i, _):
        def zcol(j, _):
            rows0[i, pl.ds(j * 16, 16)] = jnp.zeros((16,), jnp.float32)
            return 0
        return lax.fori_loop(0, D // 16, zcol, 0)
    lax.fori_loop(0, K, zrow, 0)

    def zero_chunk(z, _):
        pltpu.sync_copy(rows0, acc_s.at[pl.ds(s * ZPT + z * K, K)])
        return 0
    lax.fori_loop(0, ZPT // K, zero_chunk, 0)
    plsc.subcore_barrier()

    pltpu.sync_copy(src_hbm.at[wid, 0], sidx0)
    pltpu.async_copy(h_hbm.at[sidx0], rows0, sem0)

    def pair(i, _):
        ch2 = i * 2
        pltpu.sync_copy(src_hbm.at[wid, ch2 + 1], sidx1)
        pltpu.async_copy(h_hbm.at[sidx1], rows1, sem1)
        pltpu.make_async_copy(h_hbm.at[pl.ds(0, K)], rows0, sem0).wait()
        pltpu.sync_copy(dst_hbm.at[wid, ch2], didx_v)
        pltpu.sync_copy(rows0, acc_s.at[didx_v], add=True)

        @pl.when(ch2 + 2 < NCH)
        def _():
            pltpu.sync_copy(src_hbm.at[wid, ch2 + 2], sidx0)
            pltpu.async_copy(h_hbm.at[sidx0], rows0, sem0)
        pltpu.make_async_copy(h_hbm.at[pl.ds(0, K)], rows1, sem1).wait()
        pltpu.sync_copy(dst_hbm.at[wid, ch2 + 1], didx_v)
        pltpu.sync_copy(rows1, acc_s.at[didx_v], add=True)
        return 0
    lax.fori_loop(0, NCH // 2, pair, 0)
    plsc.subcore_barrier()

    pltpu.sync_copy(acc_s.at[pl.ds(s * ZPT, ZPT)],
                    out_hbm.at[c, pl.ds(s * ZPT, ZPT)])


_msg_kernel = pl.kernel(
    _msg_body, out_type=_MSG_OUT, mesh=_mesh, scratch_types=_MSG_SCRATCH)


def _bn(y, g, b):
    m = jnp.mean(y, axis=0, keepdims=True)
    v = jnp.mean((y - m) ** 2, axis=0, keepdims=True)
    return (y - m) / jnp.sqrt(v + 1e-5) * g + b


def _dot(a, b):
    return jnp.dot(a, b, preferred_element_type=jnp.float32)


def _dense0_body(x_ref, eap_ref,
                 Wa, ba, Wb0, bb0, g0, be0, WbA, bbA, WbB, bbB,
                 h_ref, h2a_ref, h2b_ref):
    eadeg = eap_ref[0, :N] + eap_ref[1, :N]
    ea = eadeg[:, 0:DE]
    deg = eadeg[:, DE:DE + 1]
    y = _dot(x_ref[...], Wa[...]) + ba[...]
    y = y + _dot(ea, Wb0[...]) + deg * bb0[...]
    h_ref[...] = jnp.maximum(_bn(y, g0[...], be0[...]), 0.0)
    h2a_ref[...] = _dot(ea, WbA[...]) + deg * bbA[...]
    h2b_ref[...] = _dot(ea, WbB[...]) + deg * bbB[...]


def _dense_mid_body(h1p_ref, h2_ref, h_ref,
                    W1, b1, g1, be1, W2, b2, g2, be2, o_ref):
    h1 = h1p_ref[0, :N] + h1p_ref[1, :N]
    t = jnp.maximum(
        _bn(_dot(h1, W1[...]) + b1[...] + h2_ref[...], g1[...], be1[...]), 0.0)
    o_ref[...] = jnp.maximum(
        _bn(_dot(t, W2[...]) + b2[...] + h_ref[...], g2[...], be2[...]), 0.0)


def _dense_last_body(h1p_ref, h2_ref, h_ref,
                     W1, b1, g1, be1, W2, b2, g2, be2,
                     Wl, bl, gl, bel, o_ref):
    h1 = h1p_ref[0, :N] + h1p_ref[1, :N]
    t = jnp.maximum(
        _bn(_dot(h1, W1[...]) + b1[...] + h2_ref[...], g1[...], be1[...]), 0.0)
    hb = jnp.maximum(
        _bn(_dot(t, W2[...]) + b2[...] + h_ref[...], g2[...], be2[...]), 0.0)
    o_ref[...] = _bn(_dot(hb, Wl[...]) + bl[...], gl[...], bel[...])


_f32 = jnp.float32
_dense0 = pl.pallas_call(
    _dense0_body,
    out_shape=(jax.ShapeDtypeStruct((N, D), _f32),
               jax.ShapeDtypeStruct((N, D), _f32),
               jax.ShapeDtypeStruct((N, D), _f32)))
_dense_mid = pl.pallas_call(
    _dense_mid_body, out_shape=jax.ShapeDtypeStruct((N, D), _f32))
_dense_last = pl.pallas_call(
    _dense_last_body, out_shape=jax.ShapeDtypeStruct((N, D), _f32))


def kernel(x, edge_attr, edge_index,
           W_atom, b_atom, W_bond0, b_bond0, g0, be0,
           W_bondA, b_bondA, W1A, b1A, W2A, b2A, g1A, be1A, g2A, be2A,
           W_bondB, b_bondB, W1B, b1B, W2B, b2B, g1B, be1B, g2B, be2B,
           W_last, b_last, gl, bel):
    pad = EP - E
    src = jnp.concatenate(
        [edge_index[0], jnp.zeros((pad,), jnp.int32)]).reshape(NW, NCH, K)
    dump = N + (jnp.arange(pad, dtype=jnp.int32) % (NACC - N))
    dst = jnp.concatenate([edge_index[1], dump]).reshape(NW, NCH, K)
    ea128 = jnp.concatenate(
        [edge_attr, jnp.ones((E, 1), jnp.float32),
         jnp.zeros((E, D - DE - 1), jnp.float32)], axis=1)
    ea128 = jnp.concatenate(
        [ea128, jnp.zeros((pad, D), jnp.float32)], axis=0)

    ea_parts = _ea_kernel(ea128, dst)

    r = lambda v: v.reshape(1, D)
    h, h2a, h2b = _dense0(
        x, ea_parts,
        W_atom, r(b_atom), W_bond0, r(b_bond0), r(g0), r(be0),
        W_bondA, r(b_bondA), W_bondB, r(b_bondB))

    h1p = _msg_kernel(h, src, dst)
    h = _dense_mid(h1p, h2a, h,
                   W1A, r(b1A), r(g1A), r(be1A), W2A, r(b2A), r(g2A), r(be2A))

    h1p = _msg_kernel(h, src, dst)
    out = _dense_last(h1p, h2b, h,
                      W1B, r(b1B), r(g1B), r(be1B), W2B, r(b2B), r(g2B), r(be2B),
                      W_last, r(b_last), r(gl), r(bel))
    return out


# speedup vs baseline: 8.0612x; 1.8606x over previous
import functools

import jax
import jax.numpy as jnp
from jax import lax
from jax.experimental import pallas as pl
from jax.experimental.pallas import tpu as pltpu
from jax.experimental.pallas import tpu_sc as plsc

N = 10000
E = 320000
D = 128
DE = 16

NC = 2
NS = 16
NW = NC * NS
K = 128
EPT = 10240
NCH = EPT // K
EP = NW * EPT
NACC = 10240
ZPT = NACC // NS

_mesh = plsc.VectorSubcoreMesh(
    core_axis_name="c", subcore_axis_name="s", num_cores=NC, num_subcores=NS)


_EA_OUT = jax.ShapeDtypeStruct((NC, NACC, D), jnp.float32)
_EA_SCRATCH = [
    pltpu.VMEM((K, D), jnp.float32),
    pltpu.VMEM((K, D), jnp.float32),
    pltpu.VMEM_SHARED((NACC, D), jnp.float32),
    pltpu.VMEM((K,), jnp.int32),
    pltpu.SemaphoreType.DMA,
    pltpu.SemaphoreType.DMA,
]


def _ea_body(ea_hbm, dst_hbm, out_hbm, rows0, rows1, acc_s, didx_v,
             sem0, sem1):
    c = lax.axis_index("c")
    s = lax.axis_index("s")
    wid = c * NS + s

    def zrow(i, _):
        def zcol(j, _):
            rows0[i, pl.ds(j * 16, 16)] = jnp.zeros((16,), jnp.float32)
            return 0
        return lax.fori_loop(0, D // 16, zcol, 0)
    lax.fori_loop(0, K, zrow, 0)

    def zero_chunk(z, _):
        pltpu.sync_copy(rows0, acc_s.at[pl.ds(s * ZPT + z * K, K)])
        return 0
    lax.fori_loop(0, ZPT // K, zero_chunk, 0)
    plsc.subcore_barrier()

    base = wid * EPT
    pltpu.async_copy(ea_hbm.at[pl.ds(base, K)], rows0, sem0)

    def pair(i, _):
        ch2 = i * 2
        pltpu.async_copy(ea_hbm.at[pl.ds(base + (ch2 + 1) * K, K)],
                         rows1, sem1)
        pltpu.make_async_copy(ea_hbm.at[pl.ds(0, K)], rows0, sem0).wait()
        pltpu.sync_copy(dst_hbm.at[wid, ch2], didx_v)
        pltpu.sync_copy(rows0, acc_s.at[didx_v], add=True)

        @pl.when(ch2 + 2 < NCH)
        def _():
            pltpu.async_copy(ea_hbm.at[pl.ds(base + (ch2 + 2) * K, K)],
                             rows0, sem0)
        pltpu.make_async_copy(ea_hbm.at[pl.ds(0, K)], rows1, sem1).wait()
        pltpu.sync_copy(dst_hbm.at[wid, ch2 + 1], didx_v)
        pltpu.sync_copy(rows1, acc_s.at[didx_v], add=True)
        return 0
    lax.fori_loop(0, NCH // 2, pair, 0)
    plsc.subcore_barrier()

    pltpu.sync_copy(acc_s.at[pl.ds(s * ZPT, ZPT)],
                    out_hbm.at[c, pl.ds(s * ZPT, ZPT)])


_ea_kernel = pl.kernel(
    _ea_body, out_type=_EA_OUT, mesh=_mesh, scratch_types=_EA_SCRATCH)


_MSG_OUT = jax.ShapeDtypeStruct((NC, NACC, D), jnp.float32)
_MSG_SCRATCH = [
    pltpu.VMEM((K, D), jnp.float32),
    pltpu.VMEM((K, D), jnp.float32),
    pltpu.VMEM_SHARED((NACC, D), jnp.float32),
    pltpu.VMEM((K,), jnp.int32),
    pltpu.VMEM((K,), jnp.int32),
    pltpu.VMEM((K,), jnp.int32),
    pltpu.SemaphoreType.DMA,
    pltpu.SemaphoreType.DMA,
]


def _msg_body(h_hbm, src_hbm, dst_hbm, out_hbm,
              rows0, rows1, acc_s, sidx0, sidx1, didx_v, sem0, sem1):
    c = lax.axis_index("c")
    s = lax.axis_index("s")
    wid = c * NS + s

    def zrow(i, _):
        def zcol(j, _):
            rows0[i, pl.ds(j * 16, 16)] = jnp.zeros((16,), jnp.float32)
            return 0
        return lax.fori_loop(0, D // 16, zcol, 0)
    lax.fori_loop(0, K, zrow, 0)

    def zero_chunk(z, _):
        pltpu.sync_copy(rows0, acc_s.at[pl.ds(s * ZPT + z * K, K)])
        return 0
    lax.fori_loop(0, ZPT // K, zero_chunk, 0)
    plsc.subcore_barrier()

    pltpu.sync_copy(src_hbm.at[wid, 0], sidx0)
    pltpu.async_copy(h_hbm.at[sidx0], rows0, sem0)

    def pair(i, _):
        ch2 = i * 2
        pltpu.sync_copy(src_hbm.at[wid, ch2 + 1], sidx1)
        pltpu.async_copy(h_hbm.at[sidx1], rows1, sem1)
        pltpu.make_async_copy(h_hbm.at[pl.ds(0, K)], rows0, sem0).wait()
        pltpu.sync_copy(dst_hbm.at[wid, ch2], didx_v)
        pltpu.sync_copy(rows0, acc_s.at[didx_v], add=True)

        @pl.when(ch2 + 2 < NCH)
        def _():
            pltpu.sync_copy(src_hbm.at[wid, ch2 + 2], sidx0)
            pltpu.async_copy(h_hbm.at[sidx0], rows0, sem0)
        pltpu.make_async_copy(h_hbm.at[pl.ds(0, K)], rows1, sem1).wait()
        pltpu.sync_copy(dst_hbm.at[wid, ch2 + 1], didx_v)
        pltpu.sync_copy(rows1, acc_s.at[didx_v], add=True)
        return 0
    lax.fori_loop(0, NCH // 2, pair, 0)
    plsc.subcore_barrier()

    pltpu.sync_copy(acc_s.at[pl.ds(s * ZPT, ZPT)],
                    out_hbm.at[c, pl.ds(s * ZPT, ZPT)])


_msg_kernel = pl.kernel(
    _msg_body, out_type=_MSG_OUT, mesh=_mesh, scratch_types=_MSG_SCRATCH)


def _bn(y, g, b):
    m = jnp.mean(y, axis=0, keepdims=True)
    v = jnp.mean((y - m) ** 2, axis=0, keepdims=True)
    return (y - m) / jnp.sqrt(v + 1e-5) * g + b


def _dot(a, b):
    return jnp.dot(a, b, preferred_element_type=jnp.float32)


def _dense0_body(x_ref, eap_ref,
                 Wa, ba, Wb0, bb0, g0, be0, WbA, bbA, WbB, bbB,
                 h_ref, h2a_ref, h2b_ref):
    eadeg = eap_ref[0, :N] + eap_ref[1, :N]
    ea = eadeg[:, 0:DE]
    deg = eadeg[:, DE:DE + 1]
    y = _dot(x_ref[...], Wa[...]) + ba[...]
    y = y + _dot(ea, Wb0[...]) + deg * bb0[...]
    h_ref[...] = jnp.maximum(_bn(y, g0[...], be0[...]), 0.0)
    h2a_ref[...] = _dot(ea, WbA[...]) + deg * bbA[...]
    h2b_ref[...] = _dot(ea, WbB[...]) + deg * bbB[...]


def _dense_mid_body(h1p_ref, h2_ref, h_ref,
                    W1, b1, g1, be1, W2, b2, g2, be2, o_ref):
    h1 = h1p_ref[0, :N] + h1p_ref[1, :N]
    t = jnp.maximum(
        _bn(_dot(h1, W1[...]) + b1[...] + h2_ref[...], g1[...], be1[...]), 0.0)
    o_ref[...] = jnp.maximum(
        _bn(_dot(t, W2[...]) + b2[...] + h_ref[...], g2[...], be2[...]), 0.0)


def _dense_last_body(h1p_ref, h2_ref, h_ref,
                     W1, b1, g1, be1, W2, b2, g2, be2,
                     Wl, bl, gl, bel, o_ref):
    h1 = h1p_ref[0, :N] + h1p_ref[1, :N]
    t = jnp.maximum(
        _bn(_dot(h1, W1[...]) + b1[...] + h2_ref[...], g1[...], be1[...]), 0.0)
    hb = jnp.maximum(
        _bn(_dot(t, W2[...]) + b2[...] + h_ref[...], g2[...], be2[...]), 0.0)
    o_ref[...] = _bn(_dot(hb, Wl[...]) + bl[...], gl[...], bel[...])


_f32 = jnp.float32
_dense0 = pl.pallas_call(
    _dense0_body,
    out_shape=(jax.ShapeDtypeStruct((N, D), _f32),
               jax.ShapeDtypeStruct((N, D), _f32),
               jax.ShapeDtypeStruct((N, D), _f32)))
_dense_mid = pl.pallas_call(
    _dense_mid_body, out_shape=jax.ShapeDtypeStruct((N, D), _f32))
_dense_last = pl.pallas_call(
    _dense_last_body, out_shape=jax.ShapeDtypeStruct((N, D), _f32))


def kernel(x, edge_attr, edge_index,
           W_atom, b_atom, W_bond0, b_bond0, g0, be0,
           W_bondA, b_bondA, W1A, b1A, W2A, b2A, g1A, be1A, g2A, be2A,
           W_bondB, b_bondB, W1B, b1B, W2B, b2B, g1B, be1B, g2B, be2B,
           W_last, b_last, gl, bel):
    pad = EP - E
    spread = jnp.arange(pad, dtype=jnp.int32) * 37 % N
    src = jnp.concatenate([edge_index[0], spread]).reshape(NW, NCH, K)
    dump = N + (jnp.arange(pad, dtype=jnp.int32) % (NACC - N))
    dst = jnp.concatenate([edge_index[1], dump]).reshape(NW, NCH, K)
    ea128 = jnp.concatenate(
        [edge_attr, jnp.ones((E, 1), jnp.float32),
         jnp.zeros((E, D - DE - 1), jnp.float32)], axis=1)
    ea128 = jnp.concatenate(
        [ea128, jnp.zeros((pad, D), jnp.float32)], axis=0)

    ea_parts = _ea_kernel(ea128, dst)

    r = lambda v: v.reshape(1, D)
    h, h2a, h2b = _dense0(
        x, ea_parts,
        W_atom, r(b_atom), W_bond0, r(b_bond0), r(g0), r(be0),
        W_bondA, r(b_bondA), W_bondB, r(b_bondB))

    h1p = _msg_kernel(h, src, dst)
    h = _dense_mid(h1p, h2a, h,
                   W1A, r(b1A), r(g1A), r(be1A), W2A, r(b2A), r(g2A), r(be2A))

    h1p = _msg_kernel(h, src, dst)
    out = _dense_last(h1p, h2b, h,
                      W1B, r(b1B), r(g1B), r(be1B), W2B, r(b2B), r(g2B), r(be2B),
                      W_last, r(b_last), r(gl), r(bel))
    return out


# speedup vs baseline: 8.0662x; 1.0006x over previous
import jax
import jax.numpy as jnp
from jax import lax
from jax.experimental import pallas as pl
from jax.experimental.pallas import tpu as pltpu
from jax.experimental.pallas import tpu_sc as plsc

N = 10000
E = 320000
D = 128
DE = 16

NC = 2
NS = 16
NW = NC * NS
K = 128
EPT = 10240
NCH = EPT // K
EP = NW * EPT
NACC = 10240
ZPT = NACC // NS

_mesh = plsc.VectorSubcoreMesh(
    core_axis_name="c", subcore_axis_name="s", num_cores=NC, num_subcores=NS)


_EA_OUT = jax.ShapeDtypeStruct((NC, NACC, D), jnp.float32)
_EA_SCRATCH = [
    pltpu.VMEM((K, D), jnp.float32),
    pltpu.VMEM((K, D), jnp.float32),
    pltpu.VMEM_SHARED((NACC, D), jnp.float32),
    pltpu.VMEM((K,), jnp.int32),
    pltpu.SemaphoreType.DMA,
    pltpu.SemaphoreType.DMA,
]


def _ea_body(ea_hbm, dst_hbm, out_hbm, rows0, rows1, acc_s, didx_v,
             sem0, sem1):
    c = lax.axis_index("c")
    s = lax.axis_index("s")
    wid = c * NS + s

    def zrow(i, _):
        def zcol(j, _):
            rows0[i, pl.ds(j * 16, 16)] = jnp.zeros((16,), jnp.float32)
            return 0
        return lax.fori_loop(0, D // 16, zcol, 0)
    lax.fori_loop(0, K, zrow, 0)

    def zero_chunk(z, _):
        pltpu.sync_copy(rows0, acc_s.at[pl.ds(s * ZPT + z * K, K)])
        return 0
    lax.fori_loop(0, ZPT // K, zero_chunk, 0)
    plsc.subcore_barrier()

    base = wid * EPT
    pltpu.async_copy(ea_hbm.at[pl.ds(base, K)], rows0, sem0)

    def pair(i, _):
        ch2 = i * 2
        pltpu.async_copy(ea_hbm.at[pl.ds(base + (ch2 + 1) * K, K)],
                         rows1, sem1)
        pltpu.make_async_copy(ea_hbm.at[pl.ds(0, K)], rows0, sem0).wait()
        pltpu.sync_copy(dst_hbm.at[wid, ch2], didx_v)
        pltpu.sync_copy(rows0, acc_s.at[didx_v], add=True)

        @pl.when(ch2 + 2 < NCH)
        def _():
            pltpu.async_copy(ea_hbm.at[pl.ds(base + (ch2 + 2) * K, K)],
                             rows0, sem0)
        pltpu.make_async_copy(ea_hbm.at[pl.ds(0, K)], rows1, sem1).wait()
        pltpu.sync_copy(dst_hbm.at[wid, ch2 + 1], didx_v)
        pltpu.sync_copy(rows1, acc_s.at[didx_v], add=True)
        return 0
    lax.fori_loop(0, NCH // 2, pair, 0)
    plsc.subcore_barrier()

    pltpu.sync_copy(acc_s.at[pl.ds(s * ZPT, ZPT)],
                    out_hbm.at[c, pl.ds(s * ZPT, ZPT)])


_ea_kernel = pl.kernel(
    _ea_body, out_type=_EA_OUT, mesh=_mesh, scratch_types=_EA_SCRATCH)


_MSG_OUT = jax.ShapeDtypeStruct((NC, NACC, D), jnp.float32)
_MSG_SCRATCH = [
    pltpu.VMEM((K, D), jnp.float32),
    pltpu.VMEM((K, D), jnp.float32),
    pltpu.VMEM_SHARED((NACC, D), jnp.float32),
    pltpu.VMEM((K,), jnp.int32),
    pltpu.VMEM((K,), jnp.int32),
    pltpu.VMEM((K,), jnp.int32),
    pltpu.SemaphoreType.DMA,
    pltpu.SemaphoreType.DMA,
]


def _msg_body(h_hbm, src_hbm, dst_hbm, out_hbm,
              rows0, rows1, acc_s, sidx0, sidx1, didx_v, sem0, sem1):
    c = lax.axis_index("c")
    s = lax.axis_index("s")
    wid = c * NS + s

    def zrow(i, _):
        def zcol(j, _):
            rows0[i, pl.ds(j * 16, 16)] = jnp.zeros((16,), jnp.float32)
            return 0
        return lax.fori_loop(0, D // 16, zcol, 0)
    lax.fori_loop(0, K, zrow, 0)

    def zero_chunk(z, _):
        pltpu.sync_copy(rows0, acc_s.at[pl.ds(s * ZPT + z * K, K)])
        return 0
    lax.fori_loop(0, ZPT // K, zero_chunk, 0)
    plsc.subcore_barrier()

    pltpu.sync_copy(src_hbm.at[wid, 0], sidx0)
    pltpu.async_copy(h_hbm.at[sidx0], rows0, sem0)

    def pair(i, _):
        ch2 = i * 2
        pltpu.sync_copy(src_hbm.at[wid, ch2 + 1], sidx1)
        pltpu.async_copy(h_hbm.at[sidx1], rows1, sem1)
        pltpu.make_async_copy(h_hbm.at[pl.ds(0, K)], rows0, sem0).wait()
        pltpu.sync_copy(dst_hbm.at[wid, ch2], didx_v)
        pltpu.sync_copy(rows0, acc_s.at[didx_v], add=True)

        @pl.when(ch2 + 2 < NCH)
        def _():
            pltpu.sync_copy(src_hbm.at[wid, ch2 + 2], sidx0)
            pltpu.async_copy(h_hbm.at[sidx0], rows0, sem0)
        pltpu.make_async_copy(h_hbm.at[pl.ds(0, K)], rows1, sem1).wait()
        pltpu.sync_copy(dst_hbm.at[wid, ch2 + 1], didx_v)
        pltpu.sync_copy(rows1, acc_s.at[didx_v], add=True)
        return 0
    lax.fori_loop(0, NCH // 2, pair, 0)
    plsc.subcore_barrier()

    pltpu.sync_copy(acc_s.at[pl.ds(s * ZPT, ZPT)],
                    out_hbm.at[c, pl.ds(s * ZPT, ZPT)])


_msg_kernel = pl.kernel(
    _msg_body, out_type=_MSG_OUT, mesh=_mesh, scratch_types=_MSG_SCRATCH)


def _bn(y, g, b):
    m = jnp.mean(y, axis=0, keepdims=True)
    v = jnp.mean((y - m) ** 2, axis=0, keepdims=True)
    return (y - m) / jnp.sqrt(v + 1e-5) * g + b


def _dot(a, b):
    return jnp.dot(a, b, preferred_element_type=jnp.float32)


def _dense0_body(x_ref, eap_ref,
                 Wa, ba, Wb0, bb0, g0, be0, WbA, bbA, WbB, bbB,
                 h_ref, h2a_ref, h2b_ref):
    eadeg = eap_ref[0, :N] + eap_ref[1, :N]
    ea = eadeg[:, 0:DE]
    deg = eadeg[:, DE:DE + 1]
    y = _dot(x_ref[...], Wa[...]) + ba[...]
    y = y + _dot(ea, Wb0[...]) + deg * bb0[...]
    h_ref[...] = jnp.maximum(_bn(y, g0[...], be0[...]), 0.0)
    h2a_ref[...] = _dot(ea, WbA[...]) + deg * bbA[...]
    h2b_ref[...] = _dot(ea, WbB[...]) + deg * bbB[...]


def _dense_mid_body(h1p_ref, h2_ref, h_ref,
                    W1, b1, g1, be1, W2, b2, g2, be2, o_ref):
    h1 = h1p_ref[0, :N] + h1p_ref[1, :N]
    t = jnp.maximum(
        _bn(_dot(h1, W1[...]) + b1[...] + h2_ref[...], g1[...], be1[...]), 0.0)
    o_ref[...] = jnp.maximum(
        _bn(_dot(t, W2[...]) + b2[...] + h_ref[...], g2[...], be2[...]), 0.0)


def _dense_last_body(h1p_ref, h2_ref, h_ref,
                     W1, b1, g1, be1, W2, b2, g2, be2,
                     Wl, bl, gl, bel, o_ref):
    h1 = h1p_ref[0, :N] + h1p_ref[1, :N]
    t = jnp.maximum(
        _bn(_dot(h1, W1[...]) + b1[...] + h2_ref[...], g1[...], be1[...]), 0.0)
    hb = jnp.maximum(
        _bn(_dot(t, W2[...]) + b2[...] + h_ref[...], g2[...], be2[...]), 0.0)
    o_ref[...] = _bn(_dot(hb, Wl[...]) + bl[...], gl[...], bel[...])


_f32 = jnp.float32
_dense0 = pl.pallas_call(
    _dense0_body,
    out_shape=(jax.ShapeDtypeStruct((N, D), _f32),
               jax.ShapeDtypeStruct((N, D), _f32),
               jax.ShapeDtypeStruct((N, D), _f32)))
_dense_mid = pl.pallas_call(
    _dense_mid_body, out_shape=jax.ShapeDtypeStruct((N, D), _f32))
_dense_last = pl.pallas_call(
    _dense_last_body, out_shape=jax.ShapeDtypeStruct((N, D), _f32))


def kernel(x, edge_attr, edge_index,
           W_atom, b_atom, W_bond0, b_bond0, g0, be0,
           W_bondA, b_bondA, W1A, b1A, W2A, b2A, g1A, be1A, g2A, be2A,
           W_bondB, b_bondB, W1B, b1B, W2B, b2B, g1B, be1B, g2B, be2B,
           W_last, b_last, gl, bel):
    pad = EP - E
    spread = jnp.arange(pad, dtype=jnp.int32) * 37 % N
    src = jnp.concatenate([edge_index[0], spread]).reshape(NW, NCH, K)
    dump = N + (jnp.arange(pad, dtype=jnp.int32) % (NACC - N))
    dst = jnp.concatenate([edge_index[1], dump]).reshape(NW, NCH, K)
    ea128 = jnp.concatenate(
        [edge_attr, jnp.ones((E, 1), jnp.float32),
         jnp.zeros((E, D - DE - 1), jnp.float32)], axis=1)
    ea128 = jnp.concatenate(
        [ea128, jnp.zeros((pad, D), jnp.float32)], axis=0)

    ea_parts = _ea_kernel(ea128, dst)

    r = lambda v: v.reshape(1, D)
    h, h2a, h2b = _dense0(
        x, ea_parts,
        W_atom, r(b_atom), W_bond0, r(b_bond0), r(g0), r(be0),
        W_bondA, r(b_bondA), W_bondB, r(b_bondB))

    h1p = _msg_kernel(h, src, dst)
    h = _dense_mid(h1p, h2a, h,
                   W1A, r(b1A), r(g1A), r(be1A), W2A, r(b2A), r(g2A), r(be2A))

    h1p = _msg_kernel(h, src, dst)
    out = _dense_last(h1p, h2b, h,
                      W1B, r(b1B), r(g1B), r(be1B), W2B, r(b2B), r(g2B), r(be2B),
                      W_last, r(b_last), r(gl), r(bel))
    return out
